# baseline (speedup 1.0000x reference)
import functools

import jax
import jax.numpy as jnp
from jax.experimental import pallas as pl
from jax.experimental.pallas import tpu as pltpu

NUM_TOKENS = 8192
D_MODEL = 4096
NUM_EXPERTS = 64
BLOCK_M = 512
NUM_BLOCKS = NUM_TOKENS // BLOCK_M
NBUF = 3


def _router_body(x_hbm, w_ref, o_ref, buf_ref, in_sems):
    def block_copy(i):
        slot = i % NBUF
        return pltpu.make_async_copy(
            x_hbm.at[pl.ds(i * BLOCK_M, BLOCK_M), :],
            buf_ref.at[slot],
            in_sems.at[slot],
        )

    for i in range(NBUF):
        block_copy(i).start()
    for i in range(NUM_BLOCKS):
        block_copy(i).wait()
        o_ref[pl.ds(i * BLOCK_M, BLOCK_M), :] = jax.lax.dot_general(
            buf_ref[i % NBUF].astype(jnp.bfloat16),
            w_ref[...].astype(jnp.bfloat16),
            (((1,), (1,)), ((), ())),
            preferred_element_type=jnp.float32,
        )
        if i + NBUF < NUM_BLOCKS:
            block_copy(i + NBUF).start()


@jax.jit
def kernel(x, W):
    return pl.pallas_call(
        _router_body,
        in_specs=[
            pl.BlockSpec(memory_space=pltpu.MemorySpace.HBM),
            pl.BlockSpec(memory_space=pltpu.MemorySpace.VMEM),
        ],
        out_specs=pl.BlockSpec(memory_space=pltpu.MemorySpace.VMEM),
        out_shape=jax.ShapeDtypeStruct((NUM_TOKENS, NUM_EXPERTS), jnp.float32),
        scratch_shapes=[
            pltpu.VMEM((NBUF, BLOCK_M, D_MODEL), jnp.float32),
            pltpu.SemaphoreType.DMA((NBUF,)),
        ],
        compiler_params=pltpu.CompilerParams(
            vmem_limit_bytes=100 * 1024 * 1024,
        ),
    )(x, W)


# speedup vs baseline: 1.1815x; 1.1815x over previous
import functools

import jax
import jax.numpy as jnp
from jax.experimental import pallas as pl
from jax.experimental.pallas import tpu as pltpu

NUM_TOKENS = 8192
D_MODEL = 4096
NUM_EXPERTS = 64
BLOCK_M = 512


def _matmul_body(x_ref, w_ref, o_ref):
    o_ref[...] = jax.lax.dot_general(
        w_ref[...],
        x_ref[...],
        (((1,), (1,)), ((), ())),
        preferred_element_type=jnp.float32,
    )


@jax.jit
def kernel(x, W):
    grid = (NUM_TOKENS // BLOCK_M,)
    out_t = pl.pallas_call(
        _matmul_body,
        grid=grid,
        in_specs=[
            pl.BlockSpec((BLOCK_M, D_MODEL), lambda i: (i, 0)),
            pl.BlockSpec((NUM_EXPERTS, D_MODEL), lambda i: (0, 0)),
        ],
        out_specs=pl.BlockSpec((NUM_EXPERTS, BLOCK_M), lambda i: (0, i)),
        out_shape=jax.ShapeDtypeStruct((NUM_EXPERTS, NUM_TOKENS), jnp.float32),
        compiler_params=pltpu.CompilerParams(
            dimension_semantics=("arbitrary",),
            vmem_limit_bytes=100 * 1024 * 1024,
        ),
    )(x, W)
    return out_t.T
